# baseline (speedup 1.0000x reference)
import jax
import jax.numpy as jnp
import numpy as np
from jax import lax
from jax.experimental import pallas as pl
from jax.experimental.pallas import tpu as pltpu
from jax.experimental.pallas import tpu_sc as plsc

_D = 128
_OUT = 12

_O3, _O1, _O2 = 0, 12, 16


def _body(pack_hbm, tab_hbm, out_hbm, smem, sem):
    c = lax.axis_index("c")
    pltpu.sync_copy(pack_hbm, smem)

    def half(lo):
        descs = []
        for k in range(lo, lo + _OUT // 2):
            i = k // 3
            j = k % 3
            idx = (smem[_O1 + i] * 4096 + smem[_O2 + j] * 64
                   + smem[_O3 + k]).astype(jnp.int32)
            descs.append(pltpu.async_copy(
                tab_hbm.at[pl.ds(idx, 1)],
                out_hbm.at[np.int32(j), pl.ds(np.int32(i), 1)], sem))
        for d in descs:
            d.wait()

    @pl.when(c == 0)
    def _():
        half(0)

    @pl.when(c == 1)
    def _():
        half(_OUT // 2)


def _gather12(pack, tab):
    mesh = plsc.ScalarSubcoreMesh(axis_name="c", num_cores=2)
    f = pl.kernel(
        _body,
        mesh=mesh,
        out_type=jax.ShapeDtypeStruct((3, 4, _D), jnp.float32),
        scratch_types=[
            pltpu.SMEM((32,), jnp.uint32),
            pltpu.SemaphoreType.DMA,
        ],
        compiler_params=pltpu.CompilerParams(use_tc_tiling_on_sc=False),
    )
    return f(pack, tab)


def kernel(x, index1, index2, index3):
    tab = x.reshape(-1, _D)
    zpad = jnp.zeros((4,), index3.dtype)
    cat = jnp.concatenate([
        index3.reshape(-1), index1.reshape(-1), index2.reshape(-1),
        zpad, zpad, zpad, jnp.zeros((1,), index3.dtype),
    ])
    pack = cat.astype(jnp.uint32)
    out = _gather12(pack, tab)
    return out.transpose(1, 0, 2)


# speedup vs baseline: 1.0863x; 1.0863x over previous
import jax
import jax.numpy as jnp
import numpy as np
from jax import lax
from jax.experimental import pallas as pl
from jax.experimental.pallas import tpu as pltpu
from jax.experimental.pallas import tpu_sc as plsc

_D = 128
_OUT = 12

_O3, _O1, _O2 = 0, 12, 16


def _body(pack_hbm, tab_hbm, out_hbm, smem, sem):
    pltpu.sync_copy(pack_hbm, smem)
    descs = []
    for k in range(_OUT):
        i = k // 3
        j = k % 3
        idx = (smem[_O1 + i] * 4096 + smem[_O2 + j] * 64
               + smem[_O3 + k]).astype(jnp.int32)
        descs.append(pltpu.async_copy(
            tab_hbm.at[pl.ds(idx, 1)],
            out_hbm.at[np.int32(j), pl.ds(np.int32(i), 1)], sem))
    for d in descs:
        d.wait()


def _gather12(pack, tab):
    mesh = plsc.ScalarSubcoreMesh(axis_name="c", num_cores=1)
    f = pl.kernel(
        _body,
        mesh=mesh,
        out_type=jax.ShapeDtypeStruct((3, 4, _D), jnp.float32),
        scratch_types=[
            pltpu.SMEM((32,), jnp.uint32),
            pltpu.SemaphoreType.DMA,
        ],
        compiler_params=pltpu.CompilerParams(use_tc_tiling_on_sc=False),
    )
    return f(pack, tab)


def kernel(x, index1, index2, index3):
    tab = x.reshape(-1, _D)
    zpad = jnp.zeros((4,), index3.dtype)
    cat = jnp.concatenate([
        index3.reshape(-1), index1.reshape(-1), index2.reshape(-1),
        zpad, zpad, zpad, jnp.zeros((1,), index3.dtype),
    ])
    pack = cat.astype(jnp.uint32)
    out = _gather12(pack, tab)
    return out.transpose(1, 0, 2)
